# Initial kernel scaffold; baseline (speedup 1.0000x reference)
import jax
import jax.numpy as jnp
from jax.experimental import pallas as pl


def kernel(x, edge_index, batch, W, W_self, b):
    raise NotImplementedError("write your pallas kernel here")


# speedup vs baseline: 73.0303x; 73.0303x over previous
import functools

import jax
import jax.numpy as jnp
from jax import lax
from jax.experimental import pallas as pl
from jax.experimental.pallas import tpu as pltpu
from jax.experimental.pallas import tpu_sc as plsc

N_NODES = 10000
N_EDGES = 320000
D_FEAT = 128

_NC = 2
_NS = 16
_NW = _NC * _NS
_EPW = N_EDGES // _NW
_L = 16

def _sc_mesh():
    return plsc.VectorSubcoreMesh(
        core_axis_name="c", subcore_axis_name="s", num_cores=_NC, num_subcores=_NS
    )


def _deg_body(dst_hbm, out_hbm, dst_v, acc_v):
    wid = lax.axis_index("s") * _NC + lax.axis_index("c")
    pltpu.sync_copy(dst_hbm.at[pl.ds(wid * _EPW, _EPW)], dst_v)

    zeros = jnp.zeros((_L,), jnp.float32)

    def _zero(i, c):
        acc_v[pl.ds(i * _L, _L)] = zeros
        return c

    lax.fori_loop(0, N_NODES // _L, _zero, 0, unroll=4)

    ones = jnp.ones((_L,), jnp.float32)

    def _count(i, c):
        d = dst_v[pl.ds(i * _L, _L)]
        plsc.addupdate_scatter(acc_v, [d], ones)
        return c

    lax.fori_loop(0, _EPW // _L, _count, 0, unroll=4)
    pltpu.sync_copy(acc_v, out_hbm.at[wid])


@functools.cache
def _deg_kernel():
    return pl.kernel(
        _deg_body,
        out_type=jax.ShapeDtypeStruct((_NW, N_NODES), jnp.float32),
        mesh=_sc_mesh(),
        scratch_types=[
            pltpu.VMEM((_EPW,), jnp.int32),
            pltpu.VMEM((N_NODES,), jnp.float32),
        ],
        compiler_params=pltpu.CompilerParams(needs_layout_passes=False),
    )


def _agg_body(src_hbm, dst_hbm, w_hbm, out_hbm, src_v, dst_v, w_v, acc_v):
    wid = lax.axis_index("s") * _NC + lax.axis_index("c")
    pltpu.sync_copy(src_hbm.at[pl.ds(wid * _EPW, _EPW)], src_v)
    pltpu.sync_copy(dst_hbm.at[pl.ds(wid * _EPW, _EPW)], dst_v)
    pltpu.sync_copy(w_hbm, w_v)

    zeros = jnp.zeros((_L,), jnp.float32)

    def _zero(i, c):
        acc_v[pl.ds(i * _L, _L)] = zeros
        return c

    lax.fori_loop(0, N_NODES // _L, _zero, 0, unroll=4)

    def _edge(i, c):
        s = src_v[pl.ds(i * _L, _L)]
        d = dst_v[pl.ds(i * _L, _L)]
        vals = plsc.load_gather(w_v, [s])
        plsc.addupdate_scatter(acc_v, [d], vals)
        return c

    lax.fori_loop(0, _EPW // _L, _edge, 0, unroll=4)
    pltpu.sync_copy(acc_v, out_hbm.at[wid])


@functools.cache
def _agg_kernel():
    return pl.kernel(
        _agg_body,
        out_type=jax.ShapeDtypeStruct((_NW, N_NODES), jnp.float32),
        mesh=_sc_mesh(),
        scratch_types=[
            pltpu.VMEM((_EPW,), jnp.int32),
            pltpu.VMEM((_EPW,), jnp.int32),
            pltpu.VMEM((N_NODES,), jnp.float32),
            pltpu.VMEM((N_NODES,), jnp.float32),
        ],
        compiler_params=pltpu.CompilerParams(needs_layout_passes=False),
    )


def _mid_body(x_ref, wc_ref, degp_ref, w_ref, norm_ref, z_ref):
    y2t = lax.dot_general(
        wc_ref[...], x_ref[...], (((0,), (1,)), ((), ())),
        preferred_element_type=jnp.float32,
    )
    deg = jnp.sum(degp_ref[...], axis=0, keepdims=True)
    norm = 1.0 / jnp.sqrt(jnp.clip(deg, 1.0, None))
    norm_ref[...] = norm
    w_ref[...] = y2t[0:1, :] * norm
    z_ref[...] = y2t[1:2, :]


_mid_call = pl.pallas_call(
    _mid_body,
    out_shape=(
        jax.ShapeDtypeStruct((1, N_NODES), jnp.float32),
        jax.ShapeDtypeStruct((1, N_NODES), jnp.float32),
        jax.ShapeDtypeStruct((1, N_NODES), jnp.float32),
    ),
)


def _fin_body(aggp_ref, norm_ref, z_ref, b_ref, g_ref, act_ref, lp_ref):
    agg = jnp.sum(aggp_ref[...], axis=0, keepdims=True)
    pred = norm_ref[...] * agg + z_ref[...] + b_ref[0, 0]
    m = jnp.max(pred)
    e = jnp.exp(pred - m)
    s = jnp.sum(e)
    p = e / s
    logits = jnp.log(p + 1e-20)
    t = logits + g_ref[...]
    tm = jnp.max(t)
    idx = lax.broadcasted_iota(jnp.int32, t.shape, 1)
    act = jnp.min(jnp.where(t == tm, idx, jnp.int32(N_NODES)))
    act_ref[...] = jnp.reshape(act, (1, 1))
    p_at = jnp.sum(jnp.where(idx == act, p, 0.0))
    lp_ref[...] = jnp.reshape(jnp.log(p_at), (1, 1))


_fin_call = pl.pallas_call(
    _fin_body,
    out_shape=(
        jax.ShapeDtypeStruct((1, 1), jnp.int32),
        jax.ShapeDtypeStruct((1, 1), jnp.float32),
    ),
)


def kernel(x, edge_index, batch, W, W_self, b):
    src = edge_index[0]
    dst = edge_index[1]
    wc = jnp.concatenate([W, W_self], axis=1)
    gumbel = jax.random.gumbel(jax.random.key(42), (1, N_NODES), jnp.float32)

    degp = _deg_kernel()(dst)
    wvec, norm, z = _mid_call(x, wc, degp)
    aggp = _agg_kernel()(src, dst, jnp.reshape(wvec, (N_NODES,)))
    act, lp = _fin_call(aggp, norm, z, jnp.reshape(b, (1, 1)), gumbel)

    action_index = jnp.reshape(act, (1,))
    log_prob = lp
    return action_index, log_prob


# speedup vs baseline: 90.1767x; 1.2348x over previous
import functools

import jax
import jax.numpy as jnp
from jax import lax
from jax.experimental import pallas as pl
from jax.experimental.pallas import tpu as pltpu
from jax.experimental.pallas import tpu_sc as plsc

N_NODES = 10000
N_EDGES = 320000
D_FEAT = 128

_NC = 2
_NS = 16
_NW = _NC * _NS
_EPW = N_EDGES // _NW
_L = 16

def _sc_mesh():
    return plsc.VectorSubcoreMesh(
        core_axis_name="c", subcore_axis_name="s", num_cores=_NC, num_subcores=_NS
    )


def _deg_body(ei_hbm, out_hbm, dst_v, acc_v):
    wid = lax.axis_index("s") * _NC + lax.axis_index("c")
    pltpu.sync_copy(ei_hbm.at[pl.ds(N_EDGES + wid * _EPW, _EPW)], dst_v)

    zeros = jnp.zeros((_L,), jnp.float32)

    def _zero(i, c):
        acc_v[pl.ds(i * _L, _L)] = zeros
        return c

    lax.fori_loop(0, N_NODES // _L, _zero, 0, unroll=4)

    ones = jnp.ones((_L,), jnp.float32)

    def _count(i, c):
        d = dst_v[pl.ds(i * _L, _L)]
        plsc.addupdate_scatter(acc_v, [d], ones)
        return c

    lax.fori_loop(0, _EPW // _L, _count, 0, unroll=4)
    pltpu.sync_copy(acc_v, out_hbm.at[wid])


@functools.cache
def _deg_kernel():
    return pl.kernel(
        _deg_body,
        out_type=jax.ShapeDtypeStruct((_NW, N_NODES), jnp.float32),
        mesh=_sc_mesh(),
        scratch_types=[
            pltpu.VMEM((_EPW,), jnp.int32),
            pltpu.VMEM((N_NODES,), jnp.float32),
        ],
        compiler_params=pltpu.CompilerParams(needs_layout_passes=False),
    )


def _agg_body(ei_hbm, w_hbm, out_hbm, src_v, dst_v, w_v, acc_v):
    wid = lax.axis_index("s") * _NC + lax.axis_index("c")
    pltpu.sync_copy(ei_hbm.at[pl.ds(wid * _EPW, _EPW)], src_v)
    pltpu.sync_copy(ei_hbm.at[pl.ds(N_EDGES + wid * _EPW, _EPW)], dst_v)
    pltpu.sync_copy(w_hbm.at[0], w_v)

    zeros = jnp.zeros((_L,), jnp.float32)

    def _zero(i, c):
        acc_v[pl.ds(i * _L, _L)] = zeros
        return c

    lax.fori_loop(0, N_NODES // _L, _zero, 0, unroll=4)

    def _edge(i, c):
        s = src_v[pl.ds(i * _L, _L)]
        d = dst_v[pl.ds(i * _L, _L)]
        vals = plsc.load_gather(w_v, [s])
        plsc.addupdate_scatter(acc_v, [d], vals)
        return c

    lax.fori_loop(0, _EPW // _L, _edge, 0, unroll=4)
    pltpu.sync_copy(acc_v, out_hbm.at[wid])


@functools.cache
def _agg_kernel():
    return pl.kernel(
        _agg_body,
        out_type=jax.ShapeDtypeStruct((_NW, N_NODES), jnp.float32),
        mesh=_sc_mesh(),
        scratch_types=[
            pltpu.VMEM((_EPW,), jnp.int32),
            pltpu.VMEM((_EPW,), jnp.int32),
            pltpu.VMEM((N_NODES,), jnp.float32),
            pltpu.VMEM((N_NODES,), jnp.float32),
        ],
        compiler_params=pltpu.CompilerParams(needs_layout_passes=False),
    )


def _mid_body(x_ref, w_ref_in, ws_ref_in, degp_ref, w_ref, norm_ref, z_ref):
    wc = jnp.concatenate([w_ref_in[...], ws_ref_in[...]], axis=1)
    y2t = lax.dot_general(
        wc, x_ref[...], (((0,), (1,)), ((), ())),
        preferred_element_type=jnp.float32,
    )
    deg = jnp.sum(degp_ref[...], axis=0, keepdims=True)
    norm = 1.0 / jnp.sqrt(jnp.clip(deg, 1.0, None))
    norm_ref[...] = norm
    w_ref[...] = y2t[0:1, :] * norm
    z_ref[...] = y2t[1:2, :]


_mid_call = pl.pallas_call(
    _mid_body,
    out_shape=(
        jax.ShapeDtypeStruct((1, N_NODES), jnp.float32),
        jax.ShapeDtypeStruct((1, N_NODES), jnp.float32),
        jax.ShapeDtypeStruct((1, N_NODES), jnp.float32),
    ),
)


def _fin_body(aggp_ref, norm_ref, z_ref, b_ref, g_ref, act_ref, lp_ref):
    agg = jnp.sum(aggp_ref[...], axis=0, keepdims=True)
    pred = norm_ref[...] * agg + z_ref[...] + b_ref[0, 0]
    m = jnp.max(pred)
    e = jnp.exp(pred - m)
    s = jnp.sum(e)
    p = e / s
    logits = jnp.log(p + 1e-20)
    t = logits + g_ref[...]
    tm = jnp.max(t)
    idx = lax.broadcasted_iota(jnp.int32, t.shape, 1)
    act = jnp.min(jnp.where(t == tm, idx, jnp.int32(N_NODES)))
    act_ref[...] = jnp.reshape(act, (1, 1))
    p_at = jnp.sum(jnp.where(idx == act, p, 0.0))
    lp_ref[...] = jnp.reshape(jnp.log(p_at), (1, 1))


_fin_call = pl.pallas_call(
    _fin_body,
    out_shape=(
        jax.ShapeDtypeStruct((1, 1), jnp.int32),
        jax.ShapeDtypeStruct((1, 1), jnp.float32),
    ),
)


def _np_threefry2x32(k1, k2, x0, x1):
    import numpy as np

    def rotl(v, d):
        return (v << np.uint32(d)) | (v >> np.uint32(32 - d))

    rot_a = (13, 15, 26, 6)
    rot_b = (17, 29, 16, 24)
    ks = [k1, k2, np.uint32(k1 ^ k2 ^ np.uint32(0x1BD11BDA))]
    x = [x0 + ks[0], x1 + ks[1]]

    def rounds(x, rots):
        for r in rots:
            x[0] = x[0] + x[1]
            x[1] = x[0] ^ rotl(x[1], r)
        return x

    x = rounds(x, rot_a)
    x[0] += ks[1]
    x[1] += ks[2] + np.uint32(1)
    x = rounds(x, rot_b)
    x[0] += ks[2]
    x[1] += ks[0] + np.uint32(2)
    x = rounds(x, rot_a)
    x[0] += ks[0]
    x[1] += ks[1] + np.uint32(3)
    x = rounds(x, rot_b)
    x[0] += ks[1]
    x[1] += ks[2] + np.uint32(4)
    x = rounds(x, rot_a)
    x[0] += ks[2]
    x[1] += ks[0] + np.uint32(5)
    return x


@functools.cache
def _gumbel_const():
    import numpy as np

    old = np.seterr(over="ignore")
    try:
        k1, k2 = np.uint32(0), np.uint32(42)
        idx = np.arange(N_NODES, dtype=np.uint64)
        c1 = (idx >> np.uint64(32)).astype(np.uint32)
        c2 = (idx & np.uint64(0xFFFFFFFF)).astype(np.uint32)
        b1, b2 = _np_threefry2x32(k1, k2, c1, c2)
        bits = b1 ^ b2
        float_bits = (bits >> np.uint32(9)) | np.uint32(0x3F800000)
        floats = float_bits.view(np.float32) - np.float32(1.0)
        tiny = np.float32(np.finfo(np.float32).tiny)
        span = np.float32(np.float32(1.0) - tiny)
        u = np.maximum(tiny, floats * span + tiny)
        g = -np.log(-np.log(u))
    finally:
        np.seterr(**old)
    return g.astype(np.float32).reshape(1, N_NODES)


def kernel(x, edge_index, batch, W, W_self, b):
    gumbel = jnp.asarray(_gumbel_const())

    ei_flat = jnp.reshape(edge_index, (2 * N_EDGES,))
    degp = _deg_kernel()(ei_flat)
    wvec, norm, z = _mid_call(x, W, W_self, degp)
    aggp = _agg_kernel()(ei_flat, wvec)
    act, lp = _fin_call(aggp, norm, z, jnp.reshape(b, (1, 1)), gumbel)

    action_index = jnp.reshape(act, (1,))
    log_prob = lp
    return action_index, log_prob


# speedup vs baseline: 105.2587x; 1.1673x over previous
import functools

import jax
import jax.numpy as jnp
from jax import lax
from jax.experimental import pallas as pl
from jax.experimental.pallas import tpu as pltpu
from jax.experimental.pallas import tpu_sc as plsc

N_NODES = 10000
N_EDGES = 320000
D_FEAT = 128

_NC = 2
_NS = 16
_NW = _NC * _NS
_EPW = N_EDGES // _NW
_L = 16

def _sc_mesh():
    return plsc.VectorSubcoreMesh(
        core_axis_name="c", subcore_axis_name="s", num_cores=_NC, num_subcores=_NS
    )


def _deg_body(ei_hbm, out_hbm, dst_v, acc_v):
    wid = lax.axis_index("s") * _NC + lax.axis_index("c")
    pltpu.sync_copy(ei_hbm.at[pl.ds(N_EDGES + wid * _EPW, _EPW)], dst_v)

    zeros = jnp.zeros((_L,), jnp.float32)

    @plsc.parallel_loop(0, N_NODES // _L, unroll=8)
    def _zero(i):
        acc_v[pl.ds(i * _L, _L)] = zeros

    ones = jnp.ones((_L,), jnp.float32)

    @plsc.parallel_loop(0, _EPW // _L, unroll=8)
    def _count(i):
        d = dst_v[pl.ds(i * _L, _L)]
        plsc.addupdate_scatter(acc_v, [d], ones)

    pltpu.sync_copy(acc_v, out_hbm.at[wid])


@functools.cache
def _deg_kernel():
    return pl.kernel(
        _deg_body,
        out_type=jax.ShapeDtypeStruct((_NW, N_NODES), jnp.float32),
        mesh=_sc_mesh(),
        scratch_types=[
            pltpu.VMEM((_EPW,), jnp.int32),
            pltpu.VMEM((N_NODES,), jnp.float32),
        ],
        compiler_params=pltpu.CompilerParams(needs_layout_passes=False),
    )


def _agg_body(ei_hbm, w_hbm, out_hbm, src_v, dst_v, w_v, acc_v):
    wid = lax.axis_index("s") * _NC + lax.axis_index("c")
    pltpu.sync_copy(ei_hbm.at[pl.ds(wid * _EPW, _EPW)], src_v)
    pltpu.sync_copy(ei_hbm.at[pl.ds(N_EDGES + wid * _EPW, _EPW)], dst_v)
    pltpu.sync_copy(w_hbm.at[0], w_v)

    zeros = jnp.zeros((_L,), jnp.float32)

    @plsc.parallel_loop(0, N_NODES // _L, unroll=8)
    def _zero(i):
        acc_v[pl.ds(i * _L, _L)] = zeros

    @plsc.parallel_loop(0, _EPW // _L, unroll=8)
    def _edge(i):
        s = src_v[pl.ds(i * _L, _L)]
        d = dst_v[pl.ds(i * _L, _L)]
        vals = plsc.load_gather(w_v, [s])
        plsc.addupdate_scatter(acc_v, [d], vals)

    pltpu.sync_copy(acc_v, out_hbm.at[wid])


@functools.cache
def _agg_kernel():
    return pl.kernel(
        _agg_body,
        out_type=jax.ShapeDtypeStruct((_NW, N_NODES), jnp.float32),
        mesh=_sc_mesh(),
        scratch_types=[
            pltpu.VMEM((_EPW,), jnp.int32),
            pltpu.VMEM((_EPW,), jnp.int32),
            pltpu.VMEM((N_NODES,), jnp.float32),
            pltpu.VMEM((N_NODES,), jnp.float32),
        ],
        compiler_params=pltpu.CompilerParams(needs_layout_passes=False),
    )


def _mid_body(x_ref, w_ref_in, ws_ref_in, degp_ref, w_ref, norm_ref, z_ref):
    wc = jnp.concatenate([w_ref_in[...], ws_ref_in[...]], axis=1)
    y2t = lax.dot_general(
        wc, x_ref[...], (((0,), (1,)), ((), ())),
        preferred_element_type=jnp.float32,
    )
    deg = jnp.sum(degp_ref[...], axis=0, keepdims=True)
    norm = 1.0 / jnp.sqrt(jnp.clip(deg, 1.0, None))
    norm_ref[...] = norm
    w_ref[...] = y2t[0:1, :] * norm
    z_ref[...] = y2t[1:2, :]


_mid_call = pl.pallas_call(
    _mid_body,
    out_shape=(
        jax.ShapeDtypeStruct((1, N_NODES), jnp.float32),
        jax.ShapeDtypeStruct((1, N_NODES), jnp.float32),
        jax.ShapeDtypeStruct((1, N_NODES), jnp.float32),
    ),
)


def _fin_body(aggp_ref, norm_ref, z_ref, b_ref, g_ref, act_ref, lp_ref):
    agg = jnp.sum(aggp_ref[...], axis=0, keepdims=True)
    pred = norm_ref[...] * agg + z_ref[...] + b_ref[0, 0]
    m = jnp.max(pred)
    e = jnp.exp(pred - m)
    s = jnp.sum(e)
    p = e / s
    logits = jnp.log(p + 1e-20)
    t = logits + g_ref[...]
    tm = jnp.max(t)
    idx = lax.broadcasted_iota(jnp.int32, t.shape, 1)
    act = jnp.min(jnp.where(t == tm, idx, jnp.int32(N_NODES)))
    act_ref[...] = jnp.reshape(act, (1, 1))
    p_at = jnp.sum(jnp.where(idx == act, p, 0.0))
    lp_ref[...] = jnp.reshape(jnp.log(p_at), (1, 1))


_fin_call = pl.pallas_call(
    _fin_body,
    out_shape=(
        jax.ShapeDtypeStruct((1, 1), jnp.int32),
        jax.ShapeDtypeStruct((1, 1), jnp.float32),
    ),
)


def _np_threefry2x32(k1, k2, x0, x1):
    import numpy as np

    def rotl(v, d):
        return (v << np.uint32(d)) | (v >> np.uint32(32 - d))

    rot_a = (13, 15, 26, 6)
    rot_b = (17, 29, 16, 24)
    ks = [k1, k2, np.uint32(k1 ^ k2 ^ np.uint32(0x1BD11BDA))]
    x = [x0 + ks[0], x1 + ks[1]]

    def rounds(x, rots):
        for r in rots:
            x[0] = x[0] + x[1]
            x[1] = x[0] ^ rotl(x[1], r)
        return x

    x = rounds(x, rot_a)
    x[0] += ks[1]
    x[1] += ks[2] + np.uint32(1)
    x = rounds(x, rot_b)
    x[0] += ks[2]
    x[1] += ks[0] + np.uint32(2)
    x = rounds(x, rot_a)
    x[0] += ks[0]
    x[1] += ks[1] + np.uint32(3)
    x = rounds(x, rot_b)
    x[0] += ks[1]
    x[1] += ks[2] + np.uint32(4)
    x = rounds(x, rot_a)
    x[0] += ks[2]
    x[1] += ks[0] + np.uint32(5)
    return x


@functools.cache
def _gumbel_const():
    import numpy as np

    old = np.seterr(over="ignore")
    try:
        k1, k2 = np.uint32(0), np.uint32(42)
        idx = np.arange(N_NODES, dtype=np.uint64)
        c1 = (idx >> np.uint64(32)).astype(np.uint32)
        c2 = (idx & np.uint64(0xFFFFFFFF)).astype(np.uint32)
        b1, b2 = _np_threefry2x32(k1, k2, c1, c2)
        bits = b1 ^ b2
        float_bits = (bits >> np.uint32(9)) | np.uint32(0x3F800000)
        floats = float_bits.view(np.float32) - np.float32(1.0)
        tiny = np.float32(np.finfo(np.float32).tiny)
        span = np.float32(np.float32(1.0) - tiny)
        u = np.maximum(tiny, floats * span + tiny)
        g = -np.log(-np.log(u))
    finally:
        np.seterr(**old)
    return g.astype(np.float32).reshape(1, N_NODES)


def kernel(x, edge_index, batch, W, W_self, b):
    gumbel = jnp.asarray(_gumbel_const())

    ei_flat = jnp.reshape(edge_index, (2 * N_EDGES,))
    degp = _deg_kernel()(ei_flat)
    wvec, norm, z = _mid_call(x, W, W_self, degp)
    aggp = _agg_kernel()(ei_flat, wvec)
    act, lp = _fin_call(aggp, norm, z, jnp.reshape(b, (1, 1)), gumbel)

    action_index = jnp.reshape(act, (1,))
    log_prob = lp
    return action_index, log_prob
